# baseline (speedup 1.0000x reference)
import functools

import jax
import jax.numpy as jnp
from jax import lax
from jax.experimental import pallas as pl
from jax.experimental.pallas import tpu as pltpu
from jax.experimental.pallas import tpu_sc as plsc

SEQ, B = 4096, 4
NUM_SEGMENTS = 8
EMB_DIM = 128
OUT_DIM = 1024
N_TOKENS = SEQ * B

NC, NS = 2, 16
NW = NC * NS
TOK_PER_W = N_TOKENS // NW
CH = 32
NCH = TOK_PER_W // CH
LANES = 16
VPR = OUT_DIM // LANES


def _p_kernel(table_ref, w_ref, p_ref):
    p_ref[...] = lax.dot_general(
        table_ref[...], w_ref[...],
        dimension_numbers=(((1,), (1,)), ((), ())),
        preferred_element_type=jnp.float32,
    )


def _sc_body(p_hbm, ab_hbm, out_hbm, p_loc, ab_v, buf0, buf1, w0, w1):
    wid = lax.axis_index("s") * NC + lax.axis_index("c")
    base = wid * TOK_PER_W * OUT_DIM
    pltpu.sync_copy(p_hbm, p_loc)
    pltpu.sync_copy(ab_hbm.at[pl.ds(wid * TOK_PER_W * LANES, TOK_PER_W * LANES)],
                    ab_v)

    def half(jp, c0, buf, sem):
        @pl.when(jp > 0)
        def _():
            pltpu.make_async_copy(
                buf, out_hbm.at[pl.ds(base, CH * OUT_DIM)], sem).wait()

        def fill(t, carry):
            rb = ab_v[pl.ds(t * LANES, LANES)]
            tt = t - c0
            for k in range(VPR):
                buf[pl.ds(tt * OUT_DIM + k * LANES, LANES)] = plsc.load_gather(
                    p_loc, [rb + (k * LANES)])
            return carry

        lax.fori_loop(c0, c0 + CH, fill, 0)
        pltpu.async_copy(
            buf, out_hbm.at[pl.ds(base + c0 * OUT_DIM, CH * OUT_DIM)], sem)

    def pair(jp, carry):
        half(jp, jp * (2 * CH), buf0, w0)
        half(jp, jp * (2 * CH) + CH, buf1, w1)
        return carry

    lax.fori_loop(0, NCH // 2, pair, 0)
    pltpu.make_async_copy(buf0, out_hbm.at[pl.ds(base, CH * OUT_DIM)], w0).wait()
    pltpu.make_async_copy(buf1, out_hbm.at[pl.ds(base, CH * OUT_DIM)], w1).wait()


@jax.jit
def kernel(input, align_pos, segment_ids, table, W):
    seg = segment_ids.astype(jnp.int32).reshape(N_TOKENS)
    ab = seg[:, None] * OUT_DIM + jnp.arange(LANES, dtype=jnp.int32)[None, :]
    ab = ab.reshape(N_TOKENS * LANES)
    P = pl.pallas_call(
        _p_kernel,
        out_shape=jax.ShapeDtypeStruct((NUM_SEGMENTS, OUT_DIM), jnp.float32),
    )(table, W)

    sc_expand = functools.partial(
        pl.kernel,
        out_type=jax.ShapeDtypeStruct((N_TOKENS * OUT_DIM,), jnp.float32),
        mesh=plsc.VectorSubcoreMesh(core_axis_name="c", subcore_axis_name="s"),
        compiler_params=pltpu.CompilerParams(needs_layout_passes=False),
        scratch_types=[
            pltpu.VMEM((NUM_SEGMENTS * OUT_DIM,), jnp.float32),
            pltpu.VMEM((TOK_PER_W * LANES,), jnp.int32),
            pltpu.VMEM((CH * OUT_DIM,), jnp.float32),
            pltpu.VMEM((CH * OUT_DIM,), jnp.float32),
            pltpu.SemaphoreType.DMA,
            pltpu.SemaphoreType.DMA,
        ],
    )(_sc_body)
    out = sc_expand(P.reshape(NUM_SEGMENTS * OUT_DIM), ab)
    return out.reshape(SEQ, B, OUT_DIM)


# speedup vs baseline: 1.0004x; 1.0004x over previous
import functools

import jax
import jax.numpy as jnp
from jax import lax
from jax.experimental import pallas as pl
from jax.experimental.pallas import tpu as pltpu
from jax.experimental.pallas import tpu_sc as plsc

SEQ, B = 4096, 4
NUM_SEGMENTS = 8
EMB_DIM = 128
OUT_DIM = 1024
N_TOKENS = SEQ * B

NC, NS = 2, 16
NW = NC * NS
TOK_PER_W = N_TOKENS // NW
CH = 32
NCH = TOK_PER_W // CH
LANES = 16
VPR = OUT_DIM // LANES


def _p_kernel(table_ref, w_ref, p_ref):
    p_ref[...] = lax.dot_general(
        table_ref[...], w_ref[...],
        dimension_numbers=(((1,), (1,)), ((), ())),
        preferred_element_type=jnp.float32,
    )


def _sc_body(p_hbm, ab_hbm, out_hbm, p_loc, ab_v, buf0, buf1, w0, w1):
    wid = lax.axis_index("s") * NC + lax.axis_index("c")
    base = wid * TOK_PER_W * OUT_DIM
    pltpu.sync_copy(p_hbm, p_loc)
    pltpu.sync_copy(ab_hbm.at[pl.ds(wid * TOK_PER_W * LANES, TOK_PER_W * LANES)],
                    ab_v)

    def half(jp, c0, buf, sem):
        @pl.when(jp > 0)
        def _():
            pltpu.make_async_copy(
                buf, out_hbm.at[pl.ds(base, CH * OUT_DIM)], sem).wait()

        def fill(t, carry):
            rb = ab_v[pl.ds(t * LANES, LANES)]
            tt = t - c0
            for k in range(VPR):
                buf[pl.ds(tt * OUT_DIM + k * LANES, LANES)] = plsc.load_gather(
                    p_loc, [rb + (k * LANES)])
            return carry

        lax.fori_loop(c0, c0 + CH, fill, 0)
        pltpu.async_copy(
            buf, out_hbm.at[pl.ds(base + c0 * OUT_DIM, CH * OUT_DIM)], sem)

    def pair(jp, carry):
        half(jp, jp * (2 * CH), buf0, w0)
        half(jp, jp * (2 * CH) + CH, buf1, w1)
        return carry

    lax.fori_loop(0, NCH // 2, pair, 0)
    pltpu.make_async_copy(buf0, out_hbm.at[pl.ds(base, CH * OUT_DIM)], w0).wait()
    pltpu.make_async_copy(buf1, out_hbm.at[pl.ds(base, CH * OUT_DIM)], w1).wait()


@jax.jit
def kernel(input, align_pos, segment_ids, table, W):
    seg = segment_ids.astype(jnp.int32).reshape(N_TOKENS)
    ab = seg[:, None] * OUT_DIM + jnp.arange(LANES, dtype=jnp.int32)[None, :]
    ab = ab.reshape(N_TOKENS * LANES)
    P = pl.pallas_call(
        _p_kernel,
        out_shape=jax.ShapeDtypeStruct((NUM_SEGMENTS, OUT_DIM), jnp.float32),
    )(table, W)

    sc_expand = functools.partial(
        pl.kernel,
        out_type=jax.ShapeDtypeStruct((N_TOKENS * OUT_DIM,), jnp.float32),
        mesh=plsc.VectorSubcoreMesh(core_axis_name="c", subcore_axis_name="s"),
        compiler_params=pltpu.CompilerParams(
            needs_layout_passes=False, disable_bounds_checks=True),
        scratch_types=[
            pltpu.VMEM((NUM_SEGMENTS * OUT_DIM,), jnp.float32),
            pltpu.VMEM((TOK_PER_W * LANES,), jnp.int32),
            pltpu.VMEM((CH * OUT_DIM,), jnp.float32),
            pltpu.VMEM((CH * OUT_DIM,), jnp.float32),
            pltpu.SemaphoreType.DMA,
            pltpu.SemaphoreType.DMA,
        ],
    )(_sc_body)
    out = sc_expand(P.reshape(NUM_SEGMENTS * OUT_DIM), ab)
    return out.reshape(SEQ, B, OUT_DIM)


# speedup vs baseline: 1.5045x; 1.5039x over previous
import functools

import jax
import jax.numpy as jnp
from jax import lax
from jax.experimental import pallas as pl
from jax.experimental.pallas import tpu as pltpu
from jax.experimental.pallas import tpu_sc as plsc

SEQ, B = 4096, 4
NUM_SEGMENTS = 8
EMB_DIM = 128
OUT_DIM = 1024
N_TOKENS = SEQ * B

NC, NS = 2, 16
NW = NC * NS
TOK_PER_W = N_TOKENS // NW
CH = 32
NCH = TOK_PER_W // CH
LANES = 16
VPR = OUT_DIM // LANES


def _p_kernel(table_ref, w_ref, p_ref):
    p_ref[...] = lax.dot_general(
        table_ref[...], w_ref[...],
        dimension_numbers=(((1,), (1,)), ((), ())),
        preferred_element_type=jnp.float32,
    )


def _sc_body(p_hbm, ab_hbm, out_hbm, p_loc, ab_v, buf0, buf1, w0, w1):
    wid = lax.axis_index("s") * NC + lax.axis_index("c")
    base = wid * TOK_PER_W * OUT_DIM
    pltpu.sync_copy(p_hbm, p_loc)
    pltpu.sync_copy(ab_hbm.at[pl.ds(wid * TOK_PER_W * LANES, TOK_PER_W * LANES)],
                    ab_v)

    def half(jp, c0, buf, sem):
        @pl.when(jp > 0)
        def _():
            pltpu.make_async_copy(
                buf, out_hbm.at[pl.ds(base, CH * OUT_DIM)], sem).wait()

        def fill(t, carry):
            rb = ab_v[pl.ds(t * LANES, LANES)]
            tt = t - c0
            G = 8
            for k0 in range(0, VPR, G):
                vals = [plsc.load_gather(p_loc, [rb + ((k0 + g) * LANES)])
                        for g in range(G)]
                for g in range(G):
                    buf[pl.ds(tt * OUT_DIM + (k0 + g) * LANES, LANES)] = vals[g]
            return carry

        lax.fori_loop(c0, c0 + CH, fill, 0)
        pltpu.async_copy(
            buf, out_hbm.at[pl.ds(base + c0 * OUT_DIM, CH * OUT_DIM)], sem)

    def pair(jp, carry):
        half(jp, jp * (2 * CH), buf0, w0)
        half(jp, jp * (2 * CH) + CH, buf1, w1)
        return carry

    lax.fori_loop(0, NCH // 2, pair, 0)
    pltpu.make_async_copy(buf0, out_hbm.at[pl.ds(base, CH * OUT_DIM)], w0).wait()
    pltpu.make_async_copy(buf1, out_hbm.at[pl.ds(base, CH * OUT_DIM)], w1).wait()


@jax.jit
def kernel(input, align_pos, segment_ids, table, W):
    seg = segment_ids.astype(jnp.int32).reshape(N_TOKENS)
    ab = seg[:, None] * OUT_DIM + jnp.arange(LANES, dtype=jnp.int32)[None, :]
    ab = ab.reshape(N_TOKENS * LANES)
    P = pl.pallas_call(
        _p_kernel,
        out_shape=jax.ShapeDtypeStruct((NUM_SEGMENTS, OUT_DIM), jnp.float32),
    )(table, W)

    sc_expand = functools.partial(
        pl.kernel,
        out_type=jax.ShapeDtypeStruct((N_TOKENS * OUT_DIM,), jnp.float32),
        mesh=plsc.VectorSubcoreMesh(core_axis_name="c", subcore_axis_name="s"),
        compiler_params=pltpu.CompilerParams(
            needs_layout_passes=False, disable_bounds_checks=True),
        scratch_types=[
            pltpu.VMEM((NUM_SEGMENTS * OUT_DIM,), jnp.float32),
            pltpu.VMEM((TOK_PER_W * LANES,), jnp.int32),
            pltpu.VMEM((CH * OUT_DIM,), jnp.float32),
            pltpu.VMEM((CH * OUT_DIM,), jnp.float32),
            pltpu.SemaphoreType.DMA,
            pltpu.SemaphoreType.DMA,
        ],
    )(_sc_body)
    out = sc_expand(P.reshape(NUM_SEGMENTS * OUT_DIM), ab)
    return out.reshape(SEQ, B, OUT_DIM)


# speedup vs baseline: 1.5088x; 1.0028x over previous
import functools

import jax
import jax.numpy as jnp
from jax import lax
from jax.experimental import pallas as pl
from jax.experimental.pallas import tpu as pltpu
from jax.experimental.pallas import tpu_sc as plsc

SEQ, B = 4096, 4
NUM_SEGMENTS = 8
EMB_DIM = 128
OUT_DIM = 1024
N_TOKENS = SEQ * B

NC, NS = 2, 16
NW = NC * NS
TOK_PER_W = N_TOKENS // NW
CH = 32
NCH = TOK_PER_W // CH
LANES = 16
VPR = OUT_DIM // LANES


def _p_kernel(table_ref, w_ref, p_ref):
    p_ref[...] = lax.dot_general(
        table_ref[...], w_ref[...],
        dimension_numbers=(((1,), (1,)), ((), ())),
        preferred_element_type=jnp.float32,
    )


def _sc_body(p_hbm, ab_hbm, out_hbm, p_loc, ab_v, buf0, buf1, w0, w1):
    wid = lax.axis_index("s") * NC + lax.axis_index("c")
    base = wid * TOK_PER_W * OUT_DIM
    pltpu.sync_copy(p_hbm, p_loc)
    pltpu.sync_copy(ab_hbm.at[pl.ds(wid * TOK_PER_W * LANES, TOK_PER_W * LANES)],
                    ab_v)

    def half(jp, c0, buf, sem):
        @pl.when(jp > 0)
        def _():
            pltpu.make_async_copy(
                buf, out_hbm.at[pl.ds(base, CH * OUT_DIM)], sem).wait()

        def fill(i, carry):
            tA = c0 + 2 * i
            rbA = ab_v[pl.ds(tA * LANES, LANES)]
            rbB = ab_v[pl.ds((tA + 1) * LANES, LANES)]
            offA = (2 * i) * OUT_DIM
            offB = offA + OUT_DIM
            G = 8
            for k0 in range(0, VPR, G):
                valsA = [plsc.load_gather(p_loc, [rbA + ((k0 + g) * LANES)])
                         for g in range(G)]
                valsB = [plsc.load_gather(p_loc, [rbB + ((k0 + g) * LANES)])
                         for g in range(G)]
                for g in range(G):
                    buf[pl.ds(offA + (k0 + g) * LANES, LANES)] = valsA[g]
                for g in range(G):
                    buf[pl.ds(offB + (k0 + g) * LANES, LANES)] = valsB[g]
            return carry

        lax.fori_loop(0, CH // 2, fill, 0)
        pltpu.async_copy(
            buf, out_hbm.at[pl.ds(base + c0 * OUT_DIM, CH * OUT_DIM)], sem)

    def pair(jp, carry):
        half(jp, jp * (2 * CH), buf0, w0)
        half(jp, jp * (2 * CH) + CH, buf1, w1)
        return carry

    lax.fori_loop(0, NCH // 2, pair, 0)
    pltpu.make_async_copy(buf0, out_hbm.at[pl.ds(base, CH * OUT_DIM)], w0).wait()
    pltpu.make_async_copy(buf1, out_hbm.at[pl.ds(base, CH * OUT_DIM)], w1).wait()


@jax.jit
def kernel(input, align_pos, segment_ids, table, W):
    seg = segment_ids.astype(jnp.int32).reshape(N_TOKENS)
    ab = seg[:, None] * OUT_DIM + jnp.arange(LANES, dtype=jnp.int32)[None, :]
    ab = ab.reshape(N_TOKENS * LANES)
    P = pl.pallas_call(
        _p_kernel,
        out_shape=jax.ShapeDtypeStruct((NUM_SEGMENTS, OUT_DIM), jnp.float32),
    )(table, W)

    sc_expand = functools.partial(
        pl.kernel,
        out_type=jax.ShapeDtypeStruct((N_TOKENS * OUT_DIM,), jnp.float32),
        mesh=plsc.VectorSubcoreMesh(core_axis_name="c", subcore_axis_name="s"),
        compiler_params=pltpu.CompilerParams(
            needs_layout_passes=False, disable_bounds_checks=True),
        scratch_types=[
            pltpu.VMEM((NUM_SEGMENTS * OUT_DIM,), jnp.float32),
            pltpu.VMEM((TOK_PER_W * LANES,), jnp.int32),
            pltpu.VMEM((CH * OUT_DIM,), jnp.float32),
            pltpu.VMEM((CH * OUT_DIM,), jnp.float32),
            pltpu.SemaphoreType.DMA,
            pltpu.SemaphoreType.DMA,
        ],
    )(_sc_body)
    out = sc_expand(P.reshape(NUM_SEGMENTS * OUT_DIM), ab)
    return out.reshape(SEQ, B, OUT_DIM)
